# Initial kernel scaffold; baseline (speedup 1.0000x reference)
import jax
import jax.numpy as jnp
from jax.experimental import pallas as pl


def kernel(p, x, o, W, gamma, beta):
    raise NotImplementedError("write your pallas kernel here")


# speedup vs baseline: 1.2328x; 1.2328x over previous
import functools

import jax
import jax.numpy as jnp
from jax.experimental import pallas as pl
from jax.experimental.pallas import tpu as pltpu

_EPS = 1e-5


def _cgb_kernel(x_ref, wt_ref, gamma_ref, beta_ref, out_ref,
                sum_ref, sq_ref, *, n_rows):
    ph = pl.program_id(0)
    t = pl.program_id(1)

    @pl.when(ph == 0)
    def _stats_phase():
        @pl.when(t == 0)
        def _init():
            sum_ref[...] = jnp.zeros_like(sum_ref)
            sq_ref[...] = jnp.zeros_like(sq_ref)

        h = jnp.dot(x_ref[...], wt_ref[...],
                    preferred_element_type=jnp.float32)
        sum_ref[...] += jnp.sum(h, axis=0, keepdims=True)
        sq_ref[...] += jnp.sum(h * h, axis=0, keepdims=True)

    @pl.when(ph == 1)
    def _apply_phase():
        inv_n = jnp.float32(1.0 / n_rows)
        mean = sum_ref[...] * inv_n
        var = sq_ref[...] * inv_n - mean * mean
        scale = gamma_ref[...] * jax.lax.rsqrt(var + _EPS)
        shift = beta_ref[...] - mean * scale
        h = jnp.dot(x_ref[...], wt_ref[...],
                    preferred_element_type=jnp.float32)
        out_ref[...] = jnp.maximum(h * scale + shift, 0.0)


@functools.partial(jax.jit, static_argnames=())
def kernel(p, x, o, W, gamma, beta):
    del p, o
    n, din = x.shape
    dout = W.shape[0]
    tile = 4000
    assert n % tile == 0
    num_tiles = n // tile

    wt = W.T
    gamma2 = gamma.reshape(1, dout)
    beta2 = beta.reshape(1, dout)

    out = pl.pallas_call(
        functools.partial(_cgb_kernel, n_rows=n),
        grid=(2, num_tiles),
        in_specs=[
            pl.BlockSpec((tile, din), lambda ph, t: (t, 0)),
            pl.BlockSpec((din, dout), lambda ph, t: (0, 0)),
            pl.BlockSpec((1, dout), lambda ph, t: (0, 0)),
            pl.BlockSpec((1, dout), lambda ph, t: (0, 0)),
        ],
        out_specs=pl.BlockSpec((tile, dout), lambda ph, t: (ph * t, 0)),
        out_shape=jax.ShapeDtypeStruct((n, dout), jnp.float32),
        scratch_shapes=[
            pltpu.VMEM((1, dout), jnp.float32),
            pltpu.VMEM((1, dout), jnp.float32),
        ],
        compiler_params=pltpu.CompilerParams(
            dimension_semantics=("arbitrary", "arbitrary"),
        ),
    )(x, wt, gamma2, beta2)
    return out
